# baseline (speedup 1.0000x reference)
import functools

import jax
import jax.numpy as jnp
from jax.experimental import pallas as pl

N = 262144
D = 258
H = 256
O = 5
E = 8

T = 4096


def _moe_block(x_ref, Wg_ref, bg_ref, W1_ref, b1_ref, W2_ref, b2_ref,
               W3_ref, b3_ref, W4_ref, b4_ref, out_ref):
    x = x_ref[...]
    logits = jnp.dot(x, Wg_ref[...], preferred_element_type=jnp.float32)
    logits = logits + bg_ref[...][None, :]

    i1 = jnp.argmax(logits, axis=-1)
    m1 = jnp.max(logits, axis=-1)
    eidx = jax.lax.broadcasted_iota(jnp.int32, (x.shape[0], E), 1)
    masked = jnp.where(eidx == i1[:, None], -jnp.inf, logits)
    i2 = jnp.argmax(masked, axis=-1)
    m2 = jnp.max(masked, axis=-1)
    t = jnp.exp(m2 - m1)
    g1 = 1.0 / (1.0 + t)
    g2 = t * g1

    xb = x.astype(jnp.bfloat16)
    acc = jnp.zeros((x.shape[0], O), dtype=jnp.float32)
    for e in range(E):
        h = jnp.maximum(
            jnp.dot(xb, W1_ref[e], preferred_element_type=jnp.float32)
            .astype(jnp.bfloat16) + b1_ref[e][None, :], jnp.bfloat16(0))
        h = jnp.maximum(
            jnp.dot(h, W2_ref[e], preferred_element_type=jnp.float32)
            .astype(jnp.bfloat16) + b2_ref[e][None, :], jnp.bfloat16(0))
        h = jnp.maximum(
            jnp.dot(h, W3_ref[e], preferred_element_type=jnp.float32)
            .astype(jnp.bfloat16) + b3_ref[e][None, :], jnp.bfloat16(0))
        y = (jnp.dot(h, W4_ref[e], preferred_element_type=jnp.float32)
             + b4_ref[e][None, :])
        w = g1 * (i1 == e) + g2 * (i2 == e)
        acc = acc + w[:, None] * y
    out_ref[...] = acc


def kernel(x, Wg, bg, W1, b1, W2, b2, W3, b3, W4, b4):
    W1 = W1.astype(jnp.bfloat16)
    W2 = W2.astype(jnp.bfloat16)
    W3 = W3.astype(jnp.bfloat16)
    W4 = W4.astype(jnp.bfloat16)
    b1 = b1.astype(jnp.bfloat16)
    b2 = b2.astype(jnp.bfloat16)
    b3 = b3.astype(jnp.bfloat16)
    n = x.shape[0]
    grid = (n // T,)
    full = lambda a: pl.BlockSpec(a.shape, lambda i: (0,) * a.ndim)
    return pl.pallas_call(
        _moe_block,
        grid=grid,
        in_specs=[
            pl.BlockSpec((T, D), lambda i: (i, 0)),
            full(Wg), full(bg),
            full(W1), full(b1), full(W2), full(b2),
            full(W3), full(b3), full(W4), full(b4),
        ],
        out_specs=pl.BlockSpec((T, O), lambda i: (i, 0)),
        out_shape=jax.ShapeDtypeStruct((n, O), jnp.float32),
    )(x, Wg, bg, W1, b1, W2, b2, W3, b3, W4, b4)


# speedup vs baseline: 1.0067x; 1.0067x over previous
import functools

import jax
import jax.numpy as jnp
from jax.experimental import pallas as pl

N = 262144
D = 258
H = 256
O = 5
E = 8

T = 4096


def _moe_block(x_ref, Wg_ref, bg_ref, W1_ref, b1_ref, W2_ref, b2_ref,
               W3_ref, b3_ref, W4_ref, b4_ref, out_ref):
    x = x_ref[...]
    logits = jnp.dot(x, Wg_ref[...], preferred_element_type=jnp.float32)
    logits = logits + bg_ref[...][None, :]

    i1 = jnp.argmax(logits, axis=-1)
    m1 = jnp.max(logits, axis=-1)
    eidx = jax.lax.broadcasted_iota(jnp.int32, (x.shape[0], E), 1)
    masked = jnp.where(eidx == i1[:, None], -jnp.inf, logits)
    i2 = jnp.argmax(masked, axis=-1)
    m2 = jnp.max(masked, axis=-1)
    t = jnp.exp(m2 - m1)
    g1 = 1.0 / (1.0 + t)
    g2 = t * g1

    xb = x.astype(jnp.bfloat16)
    acc = jnp.zeros((x.shape[0], O), dtype=jnp.float32)
    for e in range(E):
        h = jnp.maximum(
            jnp.dot(xb, W1_ref[e], preferred_element_type=jnp.float32)
            + b1_ref[e][None, :], 0.0).astype(jnp.bfloat16)
        h = jnp.maximum(
            jnp.dot(h, W2_ref[e], preferred_element_type=jnp.float32)
            + b2_ref[e][None, :], 0.0).astype(jnp.bfloat16)
        h = jnp.maximum(
            jnp.dot(h, W3_ref[e], preferred_element_type=jnp.float32)
            + b3_ref[e][None, :], 0.0).astype(jnp.bfloat16)
        y = (jnp.dot(h, W4_ref[e], preferred_element_type=jnp.float32)
             + b4_ref[e][None, :])
        w = g1 * (i1 == e) + g2 * (i2 == e)
        acc = acc + w[:, None] * y
    out_ref[...] = acc


def kernel(x, Wg, bg, W1, b1, W2, b2, W3, b3, W4, b4):
    W1 = W1.astype(jnp.bfloat16)
    W2 = W2.astype(jnp.bfloat16)
    W3 = W3.astype(jnp.bfloat16)
    W4 = W4.astype(jnp.bfloat16)
    n = x.shape[0]
    grid = (n // T,)
    full = lambda a: pl.BlockSpec(a.shape, lambda i: (0,) * a.ndim)
    return pl.pallas_call(
        _moe_block,
        grid=grid,
        in_specs=[
            pl.BlockSpec((T, D), lambda i: (i, 0)),
            full(Wg), full(bg),
            full(W1), full(b1), full(W2), full(b2),
            full(W3), full(b3), full(W4), full(b4),
        ],
        out_specs=pl.BlockSpec((T, O), lambda i: (i, 0)),
        out_shape=jax.ShapeDtypeStruct((n, O), jnp.float32),
    )(x, Wg, bg, W1, b1, W2, b2, W3, b3, W4, b4)
